# Initial kernel scaffold; baseline (speedup 1.0000x reference)
import jax
import jax.numpy as jnp
from jax.experimental import pallas as pl


def kernel(x, edge_index, precomp_neigh, precomp_self, connection, self_weight, neigh_weight):
    raise NotImplementedError("write your pallas kernel here")


# speedup vs baseline: 4.4199x; 4.4199x over previous
import functools

import numpy as np
import jax
import jax.numpy as jnp
from jax import lax
from jax.experimental import pallas as pl
from jax.experimental.pallas import tpu as pltpu
from jax.experimental.pallas import tpu_sc as plsc

N_NODES = 10000
N_EDGES = 160000
NPAD = 10240
XW = 128
NW = 32
NWS = 16

GC = 128
GN = (N_EDGES // GC + NW - 1) // NW
SC_C = 800
SC_CHUNKS = N_EDGES // SC_C
SC_KMAX = (SC_CHUNKS + NWS - 1) // NWS

BE = 1600
D48 = 48
DN = 960
DS = 192


def _fidx(k, kind):
    if k == 0:
        return 0
    return 2 * k - 1 if kind == 'cos' else 2 * k


def _neigh_bases():
    def new():
        return np.zeros((5, 3, 3), dtype=np.float64)

    bases = []
    k = new(); k[0, 0, 0] = 1.0; bases.append(k)
    k = new(); k[_fidx(1, 'cos'), 0, 1] = 1.0; k[_fidx(1, 'sin'), 0, 2] = 1.0; bases.append(k)
    k = new(); k[_fidx(1, 'sin'), 0, 1] = 1.0; k[_fidx(1, 'cos'), 0, 2] = -1.0; bases.append(k)
    k = new(); k[_fidx(1, 'cos'), 1, 0] = 1.0; k[_fidx(1, 'sin'), 2, 0] = 1.0; bases.append(k)
    k = new(); k[_fidx(1, 'sin'), 1, 0] = -1.0; k[_fidx(1, 'cos'), 2, 0] = 1.0; bases.append(k)
    k = new(); k[0, 1, 1] = 1.0; k[0, 2, 2] = 1.0; bases.append(k)
    k = new(); k[0, 1, 2] = -1.0; k[0, 2, 1] = 1.0; bases.append(k)
    k = new()
    k[_fidx(2, 'cos'), 1, 1] = 1.0; k[_fidx(2, 'cos'), 2, 2] = -1.0
    k[_fidx(2, 'sin'), 1, 2] = 1.0; k[_fidx(2, 'sin'), 2, 1] = 1.0
    bases.append(k)
    k = new()
    k[_fidx(2, 'cos'), 1, 2] = 1.0; k[_fidx(2, 'cos'), 2, 1] = 1.0
    k[_fidx(2, 'sin'), 1, 1] = -1.0; k[_fidx(2, 'sin'), 2, 2] = 1.0
    bases.append(k)
    return np.stack(bases).astype(np.float32)


def _self_bases():
    bases = []
    k = np.zeros((1, 3, 3)); k[0, 0, 0] = 1.0; bases.append(k)
    k = np.zeros((1, 3, 3)); k[0, 1, 1] = 1.0; k[0, 2, 2] = 1.0; bases.append(k)
    k = np.zeros((1, 3, 3)); k[0, 1, 2] = -1.0; k[0, 2, 1] = 1.0; bases.append(k)
    return np.stack(bases).astype(np.float32)


def _gather_kernel(xpad_hbm, src_hbm, dst_hbm, xi_hbm, xd_hbm,
                   siv, div, ri, rd, sem1, sem2):
    wid = lax.axis_index("s") * 2 + lax.axis_index("c")

    def body(k, _):
        cid = wid + NW * k

        @pl.when(cid < N_EDGES // GC)
        def _():
            off = cid * GC
            pltpu.sync_copy(src_hbm.at[pl.ds(off, GC)], siv)
            pltpu.sync_copy(dst_hbm.at[pl.ds(off, GC)], div)
            a = pltpu.async_copy(xpad_hbm.at[siv], ri, sem1)
            b = pltpu.async_copy(xpad_hbm.at[div], rd, sem2)
            a.wait()
            b.wait()
            pltpu.sync_copy(ri, xi_hbm.at[pl.ds(off, GC)])
            pltpu.sync_copy(rd, xd_hbm.at[pl.ds(off, GC)])

        return 0

    lax.fori_loop(0, GN, body, 0)


def _gather_xy(xpad, src, dst):
    f = functools.partial(
        pl.kernel,
        mesh=plsc.VectorSubcoreMesh(core_axis_name="c", subcore_axis_name="s"),
        out_type=(jax.ShapeDtypeStruct((N_EDGES, XW), jnp.float32),
                  jax.ShapeDtypeStruct((N_EDGES, XW), jnp.float32)),
        scratch_types=[pltpu.VMEM((GC,), jnp.int32),
                       pltpu.VMEM((GC,), jnp.int32),
                       pltpu.VMEM((GC, XW), jnp.float32),
                       pltpu.VMEM((GC, XW), jnp.float32),
                       pltpu.SemaphoreType.DMA,
                       pltpu.SemaphoreType.DMA],
    )(_gather_kernel)
    return f(xpad, src, dst)


def _tc_body(xi_ref, xd_ref, th_ref, pn_ref, ps_ref, wn_ref, ws_ref, pp_ref):
    xd = xd_ref[:, :D48]
    th = th_ref[...]
    c = jnp.cos(th)
    s = jnp.sin(th)
    a = xd[:, 16:32]
    b = xd[:, 32:48]
    xr = jnp.concatenate([xd[:, 0:16], c * a - s * b, s * a + c * b], axis=1)
    pn = pn_ref[...]
    u = jnp.concatenate([pn[:, k:k + 1] * xr for k in range(20)], axis=1)
    yd = jnp.dot(u, wn_ref[...], preferred_element_type=jnp.float32,
                 precision=lax.Precision.HIGHEST)
    xi = xi_ref[:, :D48]
    ps = ps_ref[...]
    us = jnp.concatenate([ps[:, k:k + 1] * xi for k in range(4)], axis=1)
    ys = jnp.dot(us, ws_ref[...], preferred_element_type=jnp.float32,
                 precision=lax.Precision.HIGHEST)
    prod = yd * ys
    h0 = prod[:, 0:8] + prod[:, 16:24] + prod[:, 32:40]
    h1 = prod[:, 8:16] + prod[:, 24:32] + prod[:, 40:48]
    inv = np.float32(1.0 / np.sqrt(24.0))
    p0 = jnp.sum(h0, axis=1, keepdims=True) * inv
    p1 = jnp.sum(h1, axis=1, keepdims=True) * inv
    pp_ref[...] = jnp.concatenate([p0, p1], axis=1)


def _tc_logits(xi, xd, theta, pn20, ps4, wn, ws, interpret=False):
    grid = (N_EDGES // BE,)
    return pl.pallas_call(
        _tc_body,
        grid=grid,
        in_specs=[
            pl.BlockSpec((BE, XW), lambda i: (i, 0)),
            pl.BlockSpec((BE, XW), lambda i: (i, 0)),
            pl.BlockSpec((BE, 1), lambda i: (i, 0)),
            pl.BlockSpec((BE, 20), lambda i: (i, 0)),
            pl.BlockSpec((BE, 4), lambda i: (i, 0)),
            pl.BlockSpec((DN, D48), lambda i: (0, 0)),
            pl.BlockSpec((DS, D48), lambda i: (0, 0)),
        ],
        out_specs=pl.BlockSpec((BE, 2), lambda i: (i, 0)),
        out_shape=jax.ShapeDtypeStruct((N_EDGES, 2), jnp.float32),
        interpret=interpret,
    )(xi, xd, theta, pn20, ps4, wn, ws)


def _softmax_kernel(src_hbm, pp_hbm, ap_hbm, ep_hbm,
                    tmax, tden, mbuf, macc,
                    srcv, ppv, epv, apv,
                    sh, msh, dsh):
    cid = lax.axis_index("c")
    sid = lax.axis_index("s")
    on0 = cid == 0
    NEG = jnp.float32(-3e38)
    lane = lax.iota(jnp.int32, 16)
    m2 = lane < 2
    negfill = jnp.full((16,), NEG, jnp.float32)
    zero = jnp.zeros((16,), jnp.float32)
    degv = jnp.where(lane == 2, jnp.float32(1.0), jnp.float32(0.0))

    @pl.when(on0)
    def _():
        def initm(k, _):
            tmax[pl.ds(k * 16, 16)] = negfill
            return 0

        lax.fori_loop(0, (2 * NPAD + 16) // 16, initm, 0)

        def chunk(k, _):
            ch = sid + NWS * k

            @pl.when(ch < SC_CHUNKS)
            def _():
                off = ch * SC_C
                pltpu.sync_copy(src_hbm.at[pl.ds(off, SC_C)], srcv.at[pl.ds(0, SC_C)])
                pltpu.sync_copy(pp_hbm.at[pl.ds(2 * off, 2 * SC_C)], ppv.at[pl.ds(0, 2 * SC_C)])

                def body(e, _):
                    s = srcv[pl.ds(e, 16)][0]
                    pw = ppv[pl.ds(2 * e, 16)]
                    v = jnp.where(m2, pw, negfill)
                    d = pl.ds(2 * s, 16)
                    tmax[d] = jnp.maximum(tmax[d], v)
                    return 0

                lax.fori_loop(0, SC_C, body, 0)

            return 0

        lax.fori_loop(0, SC_KMAX, chunk, 0)
        pltpu.sync_copy(tmax.at[pl.ds(0, 2 * NPAD)], sh.at[pl.ds(sid * 4 * NPAD, 2 * NPAD)])

    plsc.subcore_barrier()

    @pl.when(on0)
    def _():
        W1 = 2 * NPAD // NWS

        def initacc(k, _):
            macc[pl.ds(k * 16, 16)] = negfill
            return 0

        lax.fori_loop(0, W1 // 16, initacc, 0)

        def mstep(t, _):
            pltpu.sync_copy(sh.at[pl.ds(t * 4 * NPAD + sid * W1, W1)], mbuf.at[pl.ds(0, W1)])

            def vmax(k, _):
                d = pl.ds(k * 16, 16)
                macc[d] = jnp.maximum(macc[d], mbuf[d])
                return 0

            lax.fori_loop(0, W1 // 16, vmax, 0)
            return 0

        lax.fori_loop(0, NWS, mstep, 0)
        pltpu.sync_copy(macc.at[pl.ds(0, W1)], msh.at[pl.ds(sid * W1, W1)])

    plsc.subcore_barrier()

    @pl.when(on0)
    def _():
        pltpu.sync_copy(msh, tmax.at[pl.ds(0, 2 * NPAD)])

        def initd(k, _):
            tden[pl.ds(k * 16, 16)] = zero
            return 0

        lax.fori_loop(0, (4 * NPAD + 16) // 16, initd, 0)

        def chunk(k, _):
            ch = sid + NWS * k

            @pl.when(ch < SC_CHUNKS)
            def _():
                off = ch * SC_C
                pltpu.sync_copy(src_hbm.at[pl.ds(off, SC_C)], srcv.at[pl.ds(0, SC_C)])
                pltpu.sync_copy(pp_hbm.at[pl.ds(2 * off, 2 * SC_C)], ppv.at[pl.ds(0, 2 * SC_C)])

                def body(e, _):
                    s = srcv[pl.ds(e, 16)][0]
                    pw = ppv[pl.ds(2 * e, 16)]
                    mw = tmax[pl.ds(2 * s, 16)]
                    ew = jnp.exp(jnp.where(m2, pw - mw, zero))
                    de = pl.ds(2 * e, 16)
                    cur = epv[de]
                    epv[de] = jnp.where(m2, ew, cur)
                    v = jnp.where(m2, ew, zero) + degv
                    dt = pl.ds(4 * s, 16)
                    tden[dt] = tden[dt] + v
                    return 0

                lax.fori_loop(0, SC_C, body, 0)
                pltpu.sync_copy(epv.at[pl.ds(0, 2 * SC_C)],
                                ep_hbm.at[pl.ds(2 * off, 2 * SC_C)])

            return 0

        lax.fori_loop(0, SC_KMAX, chunk, 0)
        pltpu.sync_copy(tden.at[pl.ds(0, 4 * NPAD)], sh.at[pl.ds(sid * 4 * NPAD, 4 * NPAD)])

    plsc.subcore_barrier()

    @pl.when(on0)
    def _():
        W2 = 4 * NPAD // NWS

        def initacc(k, _):
            macc[pl.ds(k * 16, 16)] = zero
            return 0

        lax.fori_loop(0, W2 // 16, initacc, 0)

        def mstep(t, _):
            pltpu.sync_copy(sh.at[pl.ds(t * 4 * NPAD + sid * W2, W2)], mbuf.at[pl.ds(0, W2)])

            def vadd(k, _):
                d = pl.ds(k * 16, 16)
                macc[d] = macc[d] + mbuf[d]
                return 0

            lax.fori_loop(0, W2 // 16, vadd, 0)
            return 0

        lax.fori_loop(0, NWS, mstep, 0)
        pltpu.sync_copy(macc.at[pl.ds(0, W2)], dsh.at[pl.ds(sid * W2, W2)])

    plsc.subcore_barrier()

    @pl.when(on0)
    def _():
        pltpu.sync_copy(dsh, tden.at[pl.ds(0, 4 * NPAD)])
        eps = jnp.float32(1e-16)

        def chunk(k, _):
            ch = sid + NWS * k

            @pl.when(ch < SC_CHUNKS)
            def _():
                off = ch * SC_C
                pltpu.sync_copy(src_hbm.at[pl.ds(off, SC_C)], srcv.at[pl.ds(0, SC_C)])
                pltpu.sync_copy(ep_hbm.at[pl.ds(2 * off, 2 * SC_C)], epv.at[pl.ds(0, 2 * SC_C)])

                def body(e, _):
                    s = srcv[pl.ds(e, 16)][0]
                    dw = tden[pl.ds(4 * s, 16)]
                    deg = dw[2]
                    ew = epv[pl.ds(2 * e, 16)]
                    aw = deg * ew / (dw + eps)
                    de = pl.ds(2 * e, 16)
                    cur = apv[de]
                    apv[de] = jnp.where(m2, aw, cur)
                    return 0

                lax.fori_loop(0, SC_C, body, 0)
                pltpu.sync_copy(apv.at[pl.ds(0, 2 * SC_C)],
                                ap_hbm.at[pl.ds(2 * off, 2 * SC_C)])

            return 0

        lax.fori_loop(0, SC_KMAX, chunk, 0)


def _softmax(src, pp):
    f = functools.partial(
        pl.kernel,
        mesh=plsc.VectorSubcoreMesh(core_axis_name="c", subcore_axis_name="s"),
        out_type=(jax.ShapeDtypeStruct((2 * N_EDGES,), jnp.float32),
                  jax.ShapeDtypeStruct((2 * N_EDGES,), jnp.float32)),
        scratch_types=[pltpu.VMEM((2 * NPAD + 16,), jnp.float32),
                       pltpu.VMEM((4 * NPAD + 16,), jnp.float32),
                       pltpu.VMEM((4 * NPAD // NWS,), jnp.float32),
                       pltpu.VMEM((4 * NPAD // NWS,), jnp.float32),
                       pltpu.VMEM((SC_C + 16,), jnp.int32),
                       pltpu.VMEM((2 * SC_C + 16,), jnp.float32),
                       pltpu.VMEM((2 * SC_C + 16,), jnp.float32),
                       pltpu.VMEM((2 * SC_C + 16,), jnp.float32),
                       pltpu.VMEM_SHARED((NWS * 4 * NPAD,), jnp.float32),
                       pltpu.VMEM_SHARED((2 * NPAD,), jnp.float32),
                       pltpu.VMEM_SHARED((4 * NPAD,), jnp.float32)],
    )(_softmax_kernel)
    return f(src, pp)


def _weights(neigh_weight, self_weight):
    nk = jnp.asarray(_neigh_bases())
    sk = jnp.asarray(_self_bases())
    wn = jnp.einsum('bfnm,brij->frmjni', nk, neigh_weight).reshape(20, 48, 48)
    wn = wn.reshape(DN, D48)
    ws = jnp.einsum('bnm,brij->rmjni', sk[:, 0], self_weight).reshape(4, 48, 48)
    ws = ws.reshape(DS, D48)
    return wn, ws


def kernel(x, edge_index, precomp_neigh, precomp_self, connection,
           self_weight, neigh_weight):
    wn, ws = _weights(neigh_weight, self_weight)
    x48 = x.transpose(0, 2, 1).reshape(N_NODES, D48)
    xpad = jnp.zeros((N_NODES, XW), jnp.float32).at[:, :D48].set(x48)
    src = edge_index[0]
    dst = edge_index[1]
    xi, xd = _gather_xy(xpad, src, dst)
    theta = connection.reshape(N_EDGES, 1)
    pn20 = precomp_neigh[:, :5].reshape(N_EDGES, 20)
    ps4 = precomp_self.reshape(N_EDGES, 4)
    pp = _tc_logits(xi, xd, theta, pn20, ps4, wn, ws)
    ap, _ = _softmax(src, pp.reshape(2 * N_EDGES))
    return ap.reshape(N_EDGES, 2)


# speedup vs baseline: 5.1630x; 1.1681x over previous
import functools

import numpy as np
import jax
import jax.numpy as jnp
from jax import lax
from jax.experimental import pallas as pl
from jax.experimental.pallas import tpu as pltpu
from jax.experimental.pallas import tpu_sc as plsc

N_NODES = 10000
N_EDGES = 160000
NPAD = 10240
XW = 128
NW = 32
NWS = 16

GC = 128
GN = (N_EDGES // GC + NW - 1) // NW
SC_C = 800
SC_CHUNKS = N_EDGES // SC_C
SC_KMAX = (SC_CHUNKS + NWS - 1) // NWS

BE = 1600
D48 = 48
DN = 960
DS = 192


def _fidx(k, kind):
    if k == 0:
        return 0
    return 2 * k - 1 if kind == 'cos' else 2 * k


def _neigh_bases():
    def new():
        return np.zeros((5, 3, 3), dtype=np.float64)

    bases = []
    k = new(); k[0, 0, 0] = 1.0; bases.append(k)
    k = new(); k[_fidx(1, 'cos'), 0, 1] = 1.0; k[_fidx(1, 'sin'), 0, 2] = 1.0; bases.append(k)
    k = new(); k[_fidx(1, 'sin'), 0, 1] = 1.0; k[_fidx(1, 'cos'), 0, 2] = -1.0; bases.append(k)
    k = new(); k[_fidx(1, 'cos'), 1, 0] = 1.0; k[_fidx(1, 'sin'), 2, 0] = 1.0; bases.append(k)
    k = new(); k[_fidx(1, 'sin'), 1, 0] = -1.0; k[_fidx(1, 'cos'), 2, 0] = 1.0; bases.append(k)
    k = new(); k[0, 1, 1] = 1.0; k[0, 2, 2] = 1.0; bases.append(k)
    k = new(); k[0, 1, 2] = -1.0; k[0, 2, 1] = 1.0; bases.append(k)
    k = new()
    k[_fidx(2, 'cos'), 1, 1] = 1.0; k[_fidx(2, 'cos'), 2, 2] = -1.0
    k[_fidx(2, 'sin'), 1, 2] = 1.0; k[_fidx(2, 'sin'), 2, 1] = 1.0
    bases.append(k)
    k = new()
    k[_fidx(2, 'cos'), 1, 2] = 1.0; k[_fidx(2, 'cos'), 2, 1] = 1.0
    k[_fidx(2, 'sin'), 1, 1] = -1.0; k[_fidx(2, 'sin'), 2, 2] = 1.0
    bases.append(k)
    return np.stack(bases).astype(np.float32)


def _self_bases():
    bases = []
    k = np.zeros((1, 3, 3)); k[0, 0, 0] = 1.0; bases.append(k)
    k = np.zeros((1, 3, 3)); k[0, 1, 1] = 1.0; k[0, 2, 2] = 1.0; bases.append(k)
    k = np.zeros((1, 3, 3)); k[0, 1, 2] = -1.0; k[0, 2, 1] = 1.0; bases.append(k)
    return np.stack(bases).astype(np.float32)


def _gather_kernel(xpad_hbm, src_hbm, dst_hbm, xi_hbm, xd_hbm,
                   siv, div, ri, rd, sem1, sem2):
    wid = lax.axis_index("s") * 2 + lax.axis_index("c")

    def body(k, _):
        cid = wid + NW * k

        @pl.when(cid < N_EDGES // GC)
        def _():
            off = cid * GC
            pltpu.sync_copy(src_hbm.at[pl.ds(off, GC)], siv)
            pltpu.sync_copy(dst_hbm.at[pl.ds(off, GC)], div)
            a = pltpu.async_copy(xpad_hbm.at[siv], ri, sem1)
            b = pltpu.async_copy(xpad_hbm.at[div], rd, sem2)
            a.wait()
            b.wait()
            pltpu.sync_copy(ri, xi_hbm.at[pl.ds(off, GC)])
            pltpu.sync_copy(rd, xd_hbm.at[pl.ds(off, GC)])

        return 0

    lax.fori_loop(0, GN, body, 0)


def _gather_xy(xpad, src, dst):
    f = functools.partial(
        pl.kernel,
        mesh=plsc.VectorSubcoreMesh(core_axis_name="c", subcore_axis_name="s"),
        out_type=(jax.ShapeDtypeStruct((N_EDGES, XW), jnp.float32),
                  jax.ShapeDtypeStruct((N_EDGES, XW), jnp.float32)),
        scratch_types=[pltpu.VMEM((GC,), jnp.int32),
                       pltpu.VMEM((GC,), jnp.int32),
                       pltpu.VMEM((GC, XW), jnp.float32),
                       pltpu.VMEM((GC, XW), jnp.float32),
                       pltpu.SemaphoreType.DMA,
                       pltpu.SemaphoreType.DMA],
    )(_gather_kernel)
    return f(xpad, src, dst)


def _tc_body(xi_ref, xd_ref, th_ref, pn_ref, ps_ref, wn_ref, ws_ref, pp_ref):
    xd = xd_ref[:, :D48]
    th = th_ref[...]
    c = jnp.cos(th)
    s = jnp.sin(th)
    a = xd[:, 16:32]
    b = xd[:, 32:48]
    xr = jnp.concatenate([xd[:, 0:16], c * a - s * b, s * a + c * b], axis=1)
    pn = pn_ref[...]
    u = jnp.concatenate([pn[:, k:k + 1] * xr for k in range(20)], axis=1)
    yd = jnp.dot(u, wn_ref[...], preferred_element_type=jnp.float32,
                 precision=lax.Precision.DEFAULT)
    xi = xi_ref[:, :D48]
    ps = ps_ref[...]
    us = jnp.concatenate([ps[:, k:k + 1] * xi for k in range(4)], axis=1)
    ys = jnp.dot(us, ws_ref[...], preferred_element_type=jnp.float32,
                 precision=lax.Precision.DEFAULT)
    prod = yd * ys
    h0 = prod[:, 0:8] + prod[:, 16:24] + prod[:, 32:40]
    h1 = prod[:, 8:16] + prod[:, 24:32] + prod[:, 40:48]
    inv = np.float32(1.0 / np.sqrt(24.0))
    p0 = jnp.sum(h0, axis=1, keepdims=True) * inv
    p1 = jnp.sum(h1, axis=1, keepdims=True) * inv
    pp_ref[...] = jnp.concatenate([p0, p1], axis=1)


def _tc_logits(xi, xd, theta, pn20, ps4, wn, ws, interpret=False):
    grid = (N_EDGES // BE,)
    return pl.pallas_call(
        _tc_body,
        grid=grid,
        in_specs=[
            pl.BlockSpec((BE, XW), lambda i: (i, 0)),
            pl.BlockSpec((BE, XW), lambda i: (i, 0)),
            pl.BlockSpec((BE, 1), lambda i: (i, 0)),
            pl.BlockSpec((BE, 20), lambda i: (i, 0)),
            pl.BlockSpec((BE, 4), lambda i: (i, 0)),
            pl.BlockSpec((DN, D48), lambda i: (0, 0)),
            pl.BlockSpec((DS, D48), lambda i: (0, 0)),
        ],
        out_specs=pl.BlockSpec((BE, 2), lambda i: (i, 0)),
        out_shape=jax.ShapeDtypeStruct((N_EDGES, 2), jnp.float32),
        interpret=interpret,
    )(xi, xd, theta, pn20, ps4, wn, ws)


def _softmax_kernel(src_hbm, pp_hbm, ap_hbm, ep_hbm,
                    tmax, tden, mbuf, macc,
                    srcv, ppv, epv, apv,
                    sh, msh, dsh):
    cid = lax.axis_index("c")
    sid = lax.axis_index("s")
    on0 = cid == 0
    NEG = jnp.float32(-3e38)
    lane = lax.iota(jnp.int32, 16)
    m2 = lane < 2
    negfill = jnp.full((16,), NEG, jnp.float32)
    zero = jnp.zeros((16,), jnp.float32)
    degv = jnp.where(lane == 2, jnp.float32(1.0), jnp.float32(0.0))

    @pl.when(on0)
    def _():
        def initm(k, _):
            tmax[pl.ds(k * 16, 16)] = negfill
            return 0

        lax.fori_loop(0, (2 * NPAD + 16) // 16, initm, 0)

        def chunk(k, _):
            ch = sid + NWS * k

            @pl.when(ch < SC_CHUNKS)
            def _():
                off = ch * SC_C
                pltpu.sync_copy(src_hbm.at[pl.ds(off, SC_C)], srcv.at[pl.ds(0, SC_C)])
                pltpu.sync_copy(pp_hbm.at[pl.ds(2 * off, 2 * SC_C)], ppv.at[pl.ds(0, 2 * SC_C)])

                def body(e, _):
                    s = srcv[pl.ds(e, 16)][0]
                    pw = ppv[pl.ds(2 * e, 16)]
                    v = jnp.where(m2, pw, negfill)
                    d = pl.ds(2 * s, 16)
                    tmax[d] = jnp.maximum(tmax[d], v)
                    return 0

                lax.fori_loop(0, SC_C, body, 0)

            return 0

        lax.fori_loop(0, SC_KMAX, chunk, 0)
        pltpu.sync_copy(tmax.at[pl.ds(0, 2 * NPAD)], sh.at[pl.ds(sid * 4 * NPAD, 2 * NPAD)])

    plsc.subcore_barrier()

    @pl.when(on0)
    def _():
        W1 = 2 * NPAD // NWS

        def initacc(k, _):
            macc[pl.ds(k * 16, 16)] = negfill
            return 0

        lax.fori_loop(0, W1 // 16, initacc, 0)

        def mstep(t, _):
            pltpu.sync_copy(sh.at[pl.ds(t * 4 * NPAD + sid * W1, W1)], mbuf.at[pl.ds(0, W1)])

            def vmax(k, _):
                d = pl.ds(k * 16, 16)
                macc[d] = jnp.maximum(macc[d], mbuf[d])
                return 0

            lax.fori_loop(0, W1 // 16, vmax, 0)
            return 0

        lax.fori_loop(0, NWS, mstep, 0)
        pltpu.sync_copy(macc.at[pl.ds(0, W1)], msh.at[pl.ds(sid * W1, W1)])

    plsc.subcore_barrier()

    @pl.when(on0)
    def _():
        pltpu.sync_copy(msh, tmax.at[pl.ds(0, 2 * NPAD)])

        def initd(k, _):
            tden[pl.ds(k * 16, 16)] = zero
            return 0

        lax.fori_loop(0, (4 * NPAD + 16) // 16, initd, 0)

        def chunk(k, _):
            ch = sid + NWS * k

            @pl.when(ch < SC_CHUNKS)
            def _():
                off = ch * SC_C
                pltpu.sync_copy(src_hbm.at[pl.ds(off, SC_C)], srcv.at[pl.ds(0, SC_C)])
                pltpu.sync_copy(pp_hbm.at[pl.ds(2 * off, 2 * SC_C)], ppv.at[pl.ds(0, 2 * SC_C)])

                def body(e, _):
                    s = srcv[pl.ds(e, 16)][0]
                    pw = ppv[pl.ds(2 * e, 16)]
                    mw = tmax[pl.ds(2 * s, 16)]
                    ew = jnp.exp(jnp.where(m2, pw - mw, zero))
                    de = pl.ds(2 * e, 16)
                    cur = epv[de]
                    epv[de] = jnp.where(m2, ew, cur)
                    v = jnp.where(m2, ew, zero) + degv
                    dt = pl.ds(4 * s, 16)
                    tden[dt] = tden[dt] + v
                    return 0

                lax.fori_loop(0, SC_C, body, 0)
                pltpu.sync_copy(epv.at[pl.ds(0, 2 * SC_C)],
                                ep_hbm.at[pl.ds(2 * off, 2 * SC_C)])

            return 0

        lax.fori_loop(0, SC_KMAX, chunk, 0)
        pltpu.sync_copy(tden.at[pl.ds(0, 4 * NPAD)], sh.at[pl.ds(sid * 4 * NPAD, 4 * NPAD)])

    plsc.subcore_barrier()

    @pl.when(on0)
    def _():
        W2 = 4 * NPAD // NWS

        def initacc(k, _):
            macc[pl.ds(k * 16, 16)] = zero
            return 0

        lax.fori_loop(0, W2 // 16, initacc, 0)

        def mstep(t, _):
            pltpu.sync_copy(sh.at[pl.ds(t * 4 * NPAD + sid * W2, W2)], mbuf.at[pl.ds(0, W2)])

            def vadd(k, _):
                d = pl.ds(k * 16, 16)
                macc[d] = macc[d] + mbuf[d]
                return 0

            lax.fori_loop(0, W2 // 16, vadd, 0)
            return 0

        lax.fori_loop(0, NWS, mstep, 0)
        pltpu.sync_copy(macc.at[pl.ds(0, W2)], dsh.at[pl.ds(sid * W2, W2)])

    plsc.subcore_barrier()

    @pl.when(on0)
    def _():
        pltpu.sync_copy(dsh, tden.at[pl.ds(0, 4 * NPAD)])
        eps = jnp.float32(1e-16)

        def chunk(k, _):
            ch = sid + NWS * k

            @pl.when(ch < SC_CHUNKS)
            def _():
                off = ch * SC_C
                pltpu.sync_copy(src_hbm.at[pl.ds(off, SC_C)], srcv.at[pl.ds(0, SC_C)])
                pltpu.sync_copy(ep_hbm.at[pl.ds(2 * off, 2 * SC_C)], epv.at[pl.ds(0, 2 * SC_C)])

                def body(e, _):
                    s = srcv[pl.ds(e, 16)][0]
                    dw = tden[pl.ds(4 * s, 16)]
                    deg = dw[2]
                    ew = epv[pl.ds(2 * e, 16)]
                    aw = deg * ew / (dw + eps)
                    de = pl.ds(2 * e, 16)
                    cur = apv[de]
                    apv[de] = jnp.where(m2, aw, cur)
                    return 0

                lax.fori_loop(0, SC_C, body, 0)
                pltpu.sync_copy(apv.at[pl.ds(0, 2 * SC_C)],
                                ap_hbm.at[pl.ds(2 * off, 2 * SC_C)])

            return 0

        lax.fori_loop(0, SC_KMAX, chunk, 0)


def _softmax(src, pp):
    f = functools.partial(
        pl.kernel,
        mesh=plsc.VectorSubcoreMesh(core_axis_name="c", subcore_axis_name="s"),
        out_type=(jax.ShapeDtypeStruct((2 * N_EDGES,), jnp.float32),
                  jax.ShapeDtypeStruct((2 * N_EDGES,), jnp.float32)),
        scratch_types=[pltpu.VMEM((2 * NPAD + 16,), jnp.float32),
                       pltpu.VMEM((4 * NPAD + 16,), jnp.float32),
                       pltpu.VMEM((4 * NPAD // NWS,), jnp.float32),
                       pltpu.VMEM((4 * NPAD // NWS,), jnp.float32),
                       pltpu.VMEM((SC_C + 16,), jnp.int32),
                       pltpu.VMEM((2 * SC_C + 16,), jnp.float32),
                       pltpu.VMEM((2 * SC_C + 16,), jnp.float32),
                       pltpu.VMEM((2 * SC_C + 16,), jnp.float32),
                       pltpu.VMEM_SHARED((NWS * 4 * NPAD,), jnp.float32),
                       pltpu.VMEM_SHARED((2 * NPAD,), jnp.float32),
                       pltpu.VMEM_SHARED((4 * NPAD,), jnp.float32)],
    )(_softmax_kernel)
    return f(src, pp)


def _weights(neigh_weight, self_weight):
    nk = jnp.asarray(_neigh_bases())
    sk = jnp.asarray(_self_bases())
    wn = jnp.einsum('bfnm,brij->frmjni', nk, neigh_weight).reshape(20, 48, 48)
    wn = wn.reshape(DN, D48)
    ws = jnp.einsum('bnm,brij->rmjni', sk[:, 0], self_weight).reshape(4, 48, 48)
    ws = ws.reshape(DS, D48)
    return wn, ws


def kernel(x, edge_index, precomp_neigh, precomp_self, connection,
           self_weight, neigh_weight):
    wn, ws = _weights(neigh_weight, self_weight)
    x48 = x.transpose(0, 2, 1).reshape(N_NODES, D48)
    xpad = jnp.zeros((N_NODES, XW), jnp.float32).at[:, :D48].set(x48)
    src = edge_index[0]
    dst = edge_index[1]
    xi, xd = _gather_xy(xpad, src, dst)
    theta = connection.reshape(N_EDGES, 1)
    pn20 = precomp_neigh[:, :5].reshape(N_EDGES, 20)
    ps4 = precomp_self.reshape(N_EDGES, 4)
    pp = _tc_logits(xi, xd, theta, pn20, ps4, wn, ws)
    ap, _ = _softmax(src, pp.reshape(2 * N_EDGES))
    return ap.reshape(N_EDGES, 2)
